# baseline (speedup 1.0000x reference)
import functools

import jax
import jax.numpy as jnp
from jax import lax
from jax.experimental import pallas as pl
from jax.experimental.pallas import tpu as pltpu
from jax.experimental.pallas import tpu_sc as plsc

_B = 16384
_D = 64
_NC = 2
_NS = 16
_NW = _NC * _NS
_BPW = _B // _NW
_CHUNK = 128
_NCHUNK = _BPW // _CHUNK

_mesh = plsc.VectorSubcoreMesh(core_axis_name="c", subcore_axis_name="s")


@functools.partial(
    pl.kernel,
    mesh=_mesh,
    out_type=jax.ShapeDtypeStruct((_B, _D), jnp.float32),
    scratch_types=[
        pltpu.VMEM((_NCHUNK, _CHUNK), jnp.int32),
        pltpu.VMEM((_BPW, _D), jnp.float32),
        pltpu.SemaphoreType.DMA,
    ],
    compiler_params=pltpu.CompilerParams(use_tc_tiling_on_sc=False),
)
def _gather_kernel(idx_hbm, table_hbm, out_hbm, idx_v, rows_v, sem):
    wid = lax.axis_index("s") * _NC + lax.axis_index("c")
    pltpu.sync_copy(idx_hbm.at[wid], idx_v)
    copies = [
        pltpu.async_copy(
            table_hbm.at[idx_v.at[j]],
            rows_v.at[pl.ds(j * _CHUNK, _CHUNK)],
            sem,
        )
        for j in range(_NCHUNK)
    ]
    for c in copies:
        c.wait()
    pltpu.sync_copy(rows_v, out_hbm.at[pl.ds(wid * _BPW, _BPW)])


def kernel(rel, rel_emb_weight):
    idx = rel.astype(jnp.int32).reshape(_NW, _NCHUNK, _CHUNK)
    return _gather_kernel(idx, rel_emb_weight)


# speedup vs baseline: 1.0351x; 1.0351x over previous
import functools

import jax
import jax.numpy as jnp
from jax import lax
from jax.experimental import pallas as pl
from jax.experimental.pallas import tpu as pltpu
from jax.experimental.pallas import tpu_sc as plsc

_B = 16384
_D = 64
_NC = 2
_NS = 16
_NW = _NC * _NS
_BPW = _B // _NW

_mesh = plsc.VectorSubcoreMesh(core_axis_name="c", subcore_axis_name="s")


@functools.partial(
    pl.kernel,
    mesh=_mesh,
    out_type=jax.ShapeDtypeStruct((_B, _D), jnp.float32),
    scratch_types=[
        pltpu.VMEM((_BPW,), jnp.int32),
        pltpu.SemaphoreType.DMA,
    ],
)
def _gather_kernel(idx_hbm, table_hbm, out_hbm, idx_v, sem):
    wid = lax.axis_index("s") * _NC + lax.axis_index("c")
    base = wid * _BPW
    pltpu.sync_copy(idx_hbm.at[pl.ds(base, _BPW)], idx_v)

    def chunk_body(c, carry):
        k0 = c * 16
        vals = idx_v[pl.ds(k0, 16)]
        copies = []
        for l in range(16):
            i = vals[l]
            copies.append(
                pltpu.async_copy(
                    table_hbm.at[pl.ds(i, 1)],
                    out_hbm.at[pl.ds(base + k0 + l, 1)],
                    sem,
                )
            )
        for cp in copies:
            cp.wait()
        return carry

    lax.fori_loop(0, _BPW // 16, chunk_body, 0)


def kernel(rel, rel_emb_weight):
    return _gather_kernel(rel.astype(jnp.int32), rel_emb_weight)


# speedup vs baseline: 2.2962x; 2.2185x over previous
import functools

import jax
import jax.numpy as jnp
from jax import lax
from jax.experimental import pallas as pl
from jax.experimental.pallas import tpu as pltpu
from jax.experimental.pallas import tpu_sc as plsc

_B = 16384
_D = 64
_LANES = 1000000
_NC = 2
_NS = 16
_NW = _NC * _NS
_NSTRIPE = (_LANES + 127) // 128
_SPW = (_NSTRIPE + _NW - 1) // _NW
_NBUF = 3
_TRIPS = (_SPW + _NBUF - 1) // _NBUF
_LAST_W = (_NSTRIPE - 1) * 128
_LAST_N = _LANES - _LAST_W
_BATCH = 64

_mesh = plsc.VectorSubcoreMesh(core_axis_name="c", subcore_axis_name="s")


@functools.partial(
    pl.kernel,
    mesh=_mesh,
    out_type=jax.ShapeDtypeStruct((_B + _BATCH, 128), jnp.float32),
    scratch_types=[
        pltpu.VMEM((_B,), jnp.int32),
        pltpu.VMEM((_B + 16,), jnp.int32),
        pltpu.VMEM((_B + 16,), jnp.int32),
        pltpu.VMEM((_B + 16,), jnp.int32),
        pltpu.VMEM((_B + 16,), jnp.int32),
        pltpu.VMEM((_NBUF, 64, 128), jnp.float32),
        pltpu.VMEM((_BATCH, 128), jnp.float32),
        pltpu.VMEM((64, 64), jnp.float32),
        pltpu.VMEM((_BATCH,), jnp.int32),
        pltpu.SMEM((2,), jnp.int32),
        pltpu.SemaphoreType.DMA,
        pltpu.SemaphoreType.DMA,
        pltpu.SemaphoreType.DMA,
        pltpu.SemaphoreType.DMA,
        pltpu.SemaphoreType.DMA,
    ],
    compiler_params=pltpu.CompilerParams(needs_layout_passes=False),
)
def _stream_gather(idx_hbm, tw_hbm, outp_hbm, idx_v, mi, mk, smi, smk,
                   bufs, rows_v, tail_buf, klist, cnts, sem_i, sem_b0,
                   sem_b1, sem_b2, sem_fl):
    w = lax.axis_index("s") * _NC + lax.axis_index("c")
    base = w * _SPW
    nstr_w = jnp.minimum(base + _SPW, _NSTRIPE) - base
    sems = (sem_b0, sem_b1, sem_b2)
    lanes = lax.iota(jnp.int32, 16)

    pltpu.async_copy(idx_hbm, idx_v, sem_i).wait()

    for c in range(_BATCH // 16):
        klist[pl.ds(c * 16, 16)] = _B + c * 16 + lanes
    cnts[0] = 0

    lo = base * 128
    hi = (base + _SPW) * 128

    def prefilter(c, cnt):
        iv = idx_v[pl.ds(c * 16, 16)]
        m = (iv >= lo) & (iv < hi)
        mc = plsc.all_reduce_population_count(m)[0]

        @pl.when(mc > 0)
        def _():
            plsc.store_compressed(mi.at[pl.ds(cnt, 16)], iv, mask=m)
            plsc.store_compressed(mk.at[pl.ds(cnt, 16)], c * 16 + lanes, mask=m)

        return cnt + mc

    cnt = lax.fori_loop(0, _B // 16, prefilter, 0)
    nchunk = (cnt + 15) // 16

    def fire(t, b):
        sidx = base + t

        @pl.when((t < nstr_w) & (sidx < _NSTRIPE - 1))
        def _():
            pltpu.async_copy(
                tw_hbm.at[:, pl.ds(sidx * 128, 128)], bufs.at[b], sems[b])

        @pl.when((t < nstr_w) & (sidx == _NSTRIPE - 1))
        def _():
            pltpu.async_copy(
                tw_hbm.at[:, pl.ds(_LAST_W, _LAST_N)], tail_buf, sems[b])

    def wait(t, b):
        sidx = base + t

        @pl.when((t < nstr_w) & (sidx < _NSTRIPE - 1))
        def _():
            pltpu.make_async_copy(
                tw_hbm.at[:, pl.ds(sidx * 128, 128)], bufs.at[b],
                sems[b]).wait()

        @pl.when((t < nstr_w) & (sidx == _NSTRIPE - 1))
        def _():
            pltpu.make_async_copy(
                tw_hbm.at[:, pl.ds(_LAST_W, _LAST_N)], tail_buf,
                sems[b]).wait()

            def tcopy(r, z):
                for c in range(4):
                    bufs[b, r, pl.ds(c * 16, 16)] = tail_buf[r, pl.ds(c * 16, 16)]
                return z

            lax.fori_loop(0, 64, tcopy, 0)

    def process(t, b):
        @pl.when(t < nstr_w)
        def _():
            lo_s = (base + t) * 128
            cnts[1] = 0

            def scan(c, z):
                iv = mi[pl.ds(c * 16, 16)]
                kv = mk[pl.ds(c * 16, 16)]
                m = (iv >= lo_s) & (iv < lo_s + 128) & ((c * 16 + lanes) < cnt)
                mc = plsc.all_reduce_population_count(m)[0]
                sm = cnts[1]

                @pl.when(mc > 0)
                def _():
                    plsc.store_compressed(smi.at[pl.ds(sm, 16)], iv, mask=m)
                    plsc.store_compressed(smk.at[pl.ds(sm, 16)], kv, mask=m)

                cnts[1] = sm + mc
                return z

            lax.fori_loop(0, nchunk, scan, 0)

            def extract(e, z):
                eb = (e // 16) * 16
                sel = lanes == (e - eb)
                i_s = jnp.sum(jnp.where(sel, smi[pl.ds(eb, 16)], 0))
                k_s = jnp.sum(jnp.where(sel, smk[pl.ds(eb, 16)], 0))
                l_v = lax.rem(i_s, 128) + jnp.zeros((16,), jnp.int32)
                bj = cnts[0]
                for c in range(4):
                    g = plsc.load_gather(bufs.at[b], [lanes + c * 16, l_v])
                    rows_v[bj, pl.ds(c * 16, 16)] = g
                plsc.store_scatter(
                    klist, [bj + jnp.zeros((16,), jnp.int32)],
                    k_s + jnp.zeros((16,), jnp.int32), mask=lanes == 0)

                @pl.when(bj == _BATCH - 1)
                def _():
                    pltpu.async_copy(rows_v, outp_hbm.at[klist], sem_fl).wait()

                cnts[0] = lax.rem(bj + 1, _BATCH)
                return z

            lax.fori_loop(0, cnts[1], extract, 0)

    for b in range(_NBUF):
        fire(b, b)

    def outer(g, z):
        for b in range(_NBUF):
            t = g * _NBUF + b
            wait(t, b)
            process(t, b)
            fire(t + _NBUF, b)
        return z

    lax.fori_loop(0, _TRIPS, outer, 0)

    pltpu.async_copy(rows_v, outp_hbm.at[klist], sem_fl).wait()


def kernel(rel, rel_emb_weight):
    outp = _stream_gather(rel.astype(jnp.int32), rel_emb_weight.T)
    return outp[:_B, :_D]


# speedup vs baseline: 3.0451x; 1.3261x over previous
import functools

import jax
import jax.numpy as jnp
from jax import lax
from jax.experimental import pallas as pl
from jax.experimental.pallas import tpu as pltpu
from jax.experimental.pallas import tpu_sc as plsc

_B = 16384
_D = 64
_LANES = 1000000
_NC = 2
_NS = 16
_NW = _NC * _NS
_SW = 256
_NSTRIPE = (_LANES + _SW - 1) // _SW
_SPW = (_NSTRIPE + _NW - 1) // _NW
_NBUF = 2
_TRIPS = (_SPW + _NBUF - 1) // _NBUF
_LAST_W = (_NSTRIPE - 1) * _SW
_LAST_N = _LANES - _LAST_W
_BATCH = 32

_mesh = plsc.VectorSubcoreMesh(core_axis_name="c", subcore_axis_name="s")


@functools.partial(
    pl.kernel,
    mesh=_mesh,
    out_type=jax.ShapeDtypeStruct((_B + _BATCH, 128), jnp.float32),
    scratch_types=[
        pltpu.VMEM((_B,), jnp.int32),
        pltpu.VMEM((_B + 16,), jnp.int32),
        pltpu.VMEM((_B + 16,), jnp.int32),
        pltpu.VMEM((_B + 16,), jnp.int32),
        pltpu.VMEM((_B + 16,), jnp.int32),
        pltpu.VMEM((_NBUF, 64, _SW), jnp.float32),
        pltpu.VMEM((_BATCH, 128), jnp.float32),
        pltpu.VMEM((64, 64), jnp.float32),
        pltpu.VMEM((_BATCH,), jnp.int32),
        pltpu.SMEM((2,), jnp.int32),
        pltpu.SemaphoreType.DMA,
        pltpu.SemaphoreType.DMA,
        pltpu.SemaphoreType.DMA,
        pltpu.SemaphoreType.DMA,
    ],
    compiler_params=pltpu.CompilerParams(needs_layout_passes=False),
)
def _stream_gather(idx_hbm, tw_hbm, outp_hbm, idx_v, mi, mk, smi, smk,
                   bufs, rows_v, tail_buf, klist, cnts, sem_i, sem_b0,
                   sem_b1, sem_fl):
    w = lax.axis_index("s") * _NC + lax.axis_index("c")
    base = w * _SPW
    nstr_w = jnp.minimum(base + _SPW, _NSTRIPE) - base
    sems = (sem_b0, sem_b1)
    lanes = lax.iota(jnp.int32, 16)

    pltpu.async_copy(idx_hbm, idx_v, sem_i).wait()

    for c in range(_BATCH // 16):
        klist[pl.ds(c * 16, 16)] = _B + c * 16 + lanes
    cnts[0] = 0

    lo = base * _SW
    hi = (base + _SPW) * _SW

    def prefilter(c, cnt):
        iv = idx_v[pl.ds(c * 16, 16)]
        m = (iv >= lo) & (iv < hi)
        mc = plsc.all_reduce_population_count(m)[0]

        @pl.when(mc > 0)
        def _():
            plsc.store_compressed(mi.at[pl.ds(cnt, 16)], iv, mask=m)
            plsc.store_compressed(mk.at[pl.ds(cnt, 16)], c * 16 + lanes, mask=m)

        return cnt + mc

    cnt = lax.fori_loop(0, _B // 16, prefilter, 0)
    nchunk = (cnt + 15) // 16

    def fire(t, b):
        sidx = base + t

        @pl.when((t < nstr_w) & (sidx < _NSTRIPE - 1))
        def _():
            pltpu.async_copy(
                tw_hbm.at[:, pl.ds(sidx * _SW, _SW)], bufs.at[b], sems[b])

        @pl.when((t < nstr_w) & (sidx == _NSTRIPE - 1))
        def _():
            pltpu.async_copy(
                tw_hbm.at[:, pl.ds(_LAST_W, _LAST_N)], tail_buf, sems[b])

    def wait(t, b):
        sidx = base + t

        @pl.when((t < nstr_w) & (sidx < _NSTRIPE - 1))
        def _():
            pltpu.make_async_copy(
                tw_hbm.at[:, pl.ds(sidx * _SW, _SW)], bufs.at[b],
                sems[b]).wait()

        @pl.when((t < nstr_w) & (sidx == _NSTRIPE - 1))
        def _():
            pltpu.make_async_copy(
                tw_hbm.at[:, pl.ds(_LAST_W, _LAST_N)], tail_buf,
                sems[b]).wait()

            def tcopy(r, z):
                for c in range(4):
                    bufs[b, r, pl.ds(c * 16, 16)] = tail_buf[r, pl.ds(c * 16, 16)]
                return z

            lax.fori_loop(0, 64, tcopy, 0)

    def process(t, b):
        @pl.when(t < nstr_w)
        def _():
            lo_s = (base + t) * _SW
            cnts[1] = 0

            def scan(c, z):
                iv = mi[pl.ds(c * 16, 16)]
                kv = mk[pl.ds(c * 16, 16)]
                m = (iv >= lo_s) & (iv < lo_s + _SW) & ((c * 16 + lanes) < cnt)
                mc = plsc.all_reduce_population_count(m)[0]
                sm = cnts[1]

                @pl.when(mc > 0)
                def _():
                    plsc.store_compressed(smi.at[pl.ds(sm, 16)], iv, mask=m)
                    plsc.store_compressed(smk.at[pl.ds(sm, 16)], kv, mask=m)

                cnts[1] = sm + mc
                return z

            lax.fori_loop(0, nchunk, scan, 0)

            def extract(e, z):
                eb = (e // 16) * 16
                sel = lanes == (e - eb)
                i_s = jnp.sum(jnp.where(sel, smi[pl.ds(eb, 16)], 0))
                k_s = jnp.sum(jnp.where(sel, smk[pl.ds(eb, 16)], 0))
                l_v = lax.rem(i_s, _SW) + jnp.zeros((16,), jnp.int32)
                bj = cnts[0]
                for c in range(4):
                    g = plsc.load_gather(bufs.at[b], [lanes + c * 16, l_v])
                    rows_v[bj, pl.ds(c * 16, 16)] = g
                plsc.store_scatter(
                    klist, [bj + jnp.zeros((16,), jnp.int32)],
                    k_s + jnp.zeros((16,), jnp.int32), mask=lanes == 0)

                @pl.when(bj == _BATCH - 1)
                def _():
                    pltpu.async_copy(rows_v, outp_hbm.at[klist], sem_fl).wait()

                cnts[0] = lax.rem(bj + 1, _BATCH)
                return z

            lax.fori_loop(0, cnts[1], extract, 0)

    for b in range(_NBUF):
        fire(b, b)

    def outer(g, z):
        for b in range(_NBUF):
            t = g * _NBUF + b
            wait(t, b)
            process(t, b)
            fire(t + _NBUF, b)
        return z

    lax.fori_loop(0, _TRIPS, outer, 0)

    pltpu.async_copy(rows_v, outp_hbm.at[klist], sem_fl).wait()


def kernel(rel, rel_emb_weight):
    outp = _stream_gather(rel.astype(jnp.int32), rel_emb_weight.T)
    return outp[:_B, :_D]


# speedup vs baseline: 3.6962x; 1.2138x over previous
import functools

import jax
import jax.numpy as jnp
from jax import lax
from jax.experimental import pallas as pl
from jax.experimental.pallas import tpu as pltpu
from jax.experimental.pallas import tpu_sc as plsc

_B = 16384
_D = 64
_LANES = 1000000
_NC = 2
_NS = 16
_NW = _NC * _NS
_SW = 512
_NSTRIPE = (_LANES + _SW - 1) // _SW
_SPW = (_NSTRIPE + _NW - 1) // _NW
_NBUF = 2
_TRIPS = (_SPW + _NBUF - 1) // _NBUF
_LAST_W = (_NSTRIPE - 1) * _SW
_LAST_N = _LANES - _LAST_W
_BATCH = 32
_IW = 1024
_CAP = 2048

_mesh = plsc.VectorSubcoreMesh(core_axis_name="c", subcore_axis_name="s")


@functools.partial(
    pl.kernel,
    mesh=_mesh,
    out_type=jax.ShapeDtypeStruct((_B + _BATCH, 128), jnp.float32),
    scratch_types=[
        pltpu.VMEM((2, _IW), jnp.int32),
        pltpu.VMEM((_B + 16,), jnp.int32),
        pltpu.VMEM((_B + 16,), jnp.int32),
        pltpu.VMEM((_CAP + 16,), jnp.int32),
        pltpu.VMEM((_CAP + 16,), jnp.int32),
        pltpu.VMEM((_NBUF, 64, _SW), jnp.float32),
        pltpu.VMEM((_BATCH, 128), jnp.float32),
        pltpu.VMEM((64, 64), jnp.float32),
        pltpu.VMEM((_BATCH,), jnp.int32),
        pltpu.SMEM((2,), jnp.int32),
        pltpu.SemaphoreType.DMA,
        pltpu.SemaphoreType.DMA,
        pltpu.SemaphoreType.DMA,
        pltpu.SemaphoreType.DMA,
        pltpu.SemaphoreType.DMA,
    ],
    compiler_params=pltpu.CompilerParams(needs_layout_passes=False),
)
def _stream_gather(idx_hbm, tw_hbm, outp_hbm, idxw, mi, mk, smi, smk,
                   bufs, rows_v, tail_buf, klist, cnts, sem_i0, sem_i1,
                   sem_b0, sem_b1, sem_fl):
    w = lax.axis_index("s") * _NC + lax.axis_index("c")
    base = w * _SPW
    nstr_w = jnp.minimum(base + _SPW, _NSTRIPE) - base
    sems = (sem_b0, sem_b1)
    isems = (sem_i0, sem_i1)
    lanes = lax.iota(jnp.int32, 16)

    for c in range(_BATCH // 16):
        klist[pl.ds(c * 16, 16)] = _B + c * 16 + lanes
    cnts[0] = 0

    lo = base * _SW
    hi = (base + _SPW) * _SW

    for p in range(2):
        pltpu.async_copy(idx_hbm.at[pl.ds(p * _IW, _IW)], idxw.at[p],
                         isems[p])

    def prefilter_win(g, cnt):
        for p in range(2):
            win = g * 2 + p
            w0 = win * _IW
            pltpu.make_async_copy(idx_hbm.at[pl.ds(w0, _IW)], idxw.at[p],
                                  isems[p]).wait()

            def pf(c, cnt2):
                iv = idxw[p, pl.ds(c * 16, 16)]
                m = (iv >= lo) & (iv < hi)
                mc = plsc.all_reduce_population_count(m)[0]

                @pl.when(mc > 0)
                def _():
                    plsc.store_compressed(mi.at[pl.ds(cnt2, 16)], iv, mask=m)
                    plsc.store_compressed(mk.at[pl.ds(cnt2, 16)],
                                          w0 + c * 16 + lanes, mask=m)

                return cnt2 + mc

            cnt = lax.fori_loop(0, _IW // 16, pf, cnt)

            @pl.when(win + 2 < _B // _IW)
            def _():
                pltpu.async_copy(idx_hbm.at[pl.ds(w0 + 2 * _IW, _IW)],
                                 idxw.at[p], isems[p])

        return cnt

    cnt = lax.fori_loop(0, _B // _IW // 2, prefilter_win, 0)
    nchunk = (cnt + 15) // 16

    def fire(t, b):
        sidx = base + t

        @pl.when((t < nstr_w) & (sidx < _NSTRIPE - 1))
        def _():
            pltpu.async_copy(
                tw_hbm.at[:, pl.ds(sidx * _SW, _SW)], bufs.at[b], sems[b])

        @pl.when((t < nstr_w) & (sidx == _NSTRIPE - 1))
        def _():
            pltpu.async_copy(
                tw_hbm.at[:, pl.ds(_LAST_W, _LAST_N)], tail_buf, sems[b])

    def wait(t, b):
        sidx = base + t

        @pl.when((t < nstr_w) & (sidx < _NSTRIPE - 1))
        def _():
            pltpu.make_async_copy(
                tw_hbm.at[:, pl.ds(sidx * _SW, _SW)], bufs.at[b],
                sems[b]).wait()

        @pl.when((t < nstr_w) & (sidx == _NSTRIPE - 1))
        def _():
            pltpu.make_async_copy(
                tw_hbm.at[:, pl.ds(_LAST_W, _LAST_N)], tail_buf,
                sems[b]).wait()

            def tcopy(r, z):
                for c in range(4):
                    bufs[b, r, pl.ds(c * 16, 16)] = tail_buf[r, pl.ds(c * 16, 16)]
                return z

            lax.fori_loop(0, 64, tcopy, 0)

    def process(t, b):
        @pl.when(t < nstr_w)
        def _():
            lo_s = (base + t) * _SW

            def extract_all():
                def extract(e, z):
                    eb = (e // 16) * 16
                    sel = lanes == (e - eb)
                    i_s = jnp.sum(jnp.where(sel, smi[pl.ds(eb, 16)], 0))
                    k_s = jnp.sum(jnp.where(sel, smk[pl.ds(eb, 16)], 0))
                    l_v = lax.rem(i_s, _SW) + jnp.zeros((16,), jnp.int32)
                    bj = cnts[0]
                    for c in range(4):
                        g = plsc.load_gather(bufs.at[b], [lanes + c * 16, l_v])
                        rows_v[bj, pl.ds(c * 16, 16)] = g
                    plsc.store_scatter(
                        klist, [bj + jnp.zeros((16,), jnp.int32)],
                        k_s + jnp.zeros((16,), jnp.int32), mask=lanes == 0)

                    @pl.when(bj == _BATCH - 1)
                    def _():
                        pltpu.async_copy(rows_v, outp_hbm.at[klist],
                                         sem_fl).wait()

                    cnts[0] = lax.rem(bj + 1, _BATCH)
                    return z

                lax.fori_loop(0, cnts[1], extract, 0)
                cnts[1] = 0

            cnts[1] = 0

            def scan(c, z):
                @pl.when(cnts[1] + 16 > _CAP)
                def _():
                    extract_all()

                iv = mi[pl.ds(c * 16, 16)]
                kv = mk[pl.ds(c * 16, 16)]
                m = (iv >= lo_s) & (iv < lo_s + _SW) & ((c * 16 + lanes) < cnt)
                mc = plsc.all_reduce_population_count(m)[0]
                sm = cnts[1]

                @pl.when(mc > 0)
                def _():
                    plsc.store_compressed(smi.at[pl.ds(sm, 16)], iv, mask=m)
                    plsc.store_compressed(smk.at[pl.ds(sm, 16)], kv, mask=m)

                cnts[1] = sm + mc
                return z

            lax.fori_loop(0, nchunk, scan, 0)
            extract_all()

    for b in range(_NBUF):
        fire(b, b)

    def outer(g, z):
        for b in range(_NBUF):
            t = g * _NBUF + b
            wait(t, b)
            process(t, b)
            fire(t + _NBUF, b)
        return z

    lax.fori_loop(0, _TRIPS, outer, 0)

    pltpu.async_copy(rows_v, outp_hbm.at[klist], sem_fl).wait()


def kernel(rel, rel_emb_weight):
    outp = _stream_gather(rel.astype(jnp.int32), rel_emb_weight.T)
    return outp[:_B, :_D]


# speedup vs baseline: 3.7148x; 1.0050x over previous
import functools

import jax
import jax.numpy as jnp
from jax import lax
from jax.experimental import pallas as pl
from jax.experimental.pallas import tpu as pltpu
from jax.experimental.pallas import tpu_sc as plsc

_B = 16384
_D = 64
_LANES = 1000000
_NC = 2
_NS = 16
_NW = _NC * _NS
_SW = 512
_NSTRIPE = (_LANES + _SW - 1) // _SW
_SPW = (_NSTRIPE + _NW - 1) // _NW
_NBUF = 2
_TRIPS = (_SPW + _NBUF - 1) // _NBUF
_LAST_W = (_NSTRIPE - 1) * _SW
_LAST_N = _LANES - _LAST_W
_BATCH = 32
_IW = 1024
_CAP = 2048

_mesh = plsc.VectorSubcoreMesh(core_axis_name="c", subcore_axis_name="s")


@functools.partial(
    pl.kernel,
    mesh=_mesh,
    out_type=jax.ShapeDtypeStruct((_B + _BATCH, 128), jnp.float32),
    scratch_types=[
        pltpu.VMEM((2, _IW), jnp.int32),
        pltpu.VMEM((_B + 16,), jnp.int32),
        pltpu.VMEM((_B + 16,), jnp.int32),
        pltpu.VMEM((_CAP + 16,), jnp.int32),
        pltpu.VMEM((_CAP + 16,), jnp.int32),
        pltpu.VMEM((_NBUF, 64, _SW), jnp.float32),
        pltpu.VMEM((_BATCH, 128), jnp.float32),
        pltpu.VMEM((64, 64), jnp.float32),
        pltpu.VMEM((_BATCH,), jnp.int32),
        pltpu.SMEM((2,), jnp.int32),
        pltpu.SemaphoreType.DMA,
        pltpu.SemaphoreType.DMA,
        pltpu.SemaphoreType.DMA,
        pltpu.SemaphoreType.DMA,
        pltpu.SemaphoreType.DMA,
        pltpu.SemaphoreType.DMA,
        pltpu.SemaphoreType.DMA,
    ],
    compiler_params=pltpu.CompilerParams(needs_layout_passes=False),
)
def _stream_gather(idx_hbm, tw_hbm, outp_hbm, idxw, mi, mk, smi, smk,
                   bufs, rows_v, tail_buf, klist, cnts, sem_i0, sem_i1,
                   sem_b0a, sem_b0b, sem_b1a, sem_b1b, sem_fl):
    w = lax.axis_index("s") * _NC + lax.axis_index("c")
    base = w * _SPW
    nstr_w = jnp.minimum(base + _SPW, _NSTRIPE) - base
    sems = (sem_b0a, sem_b1a)
    sems2 = (sem_b0b, sem_b1b)
    isems = (sem_i0, sem_i1)
    lanes = lax.iota(jnp.int32, 16)

    for c in range(_BATCH // 16):
        klist[pl.ds(c * 16, 16)] = _B + c * 16 + lanes
    cnts[0] = 0

    def fire(t, b):
        sidx = base + t

        @pl.when((t < nstr_w) & (sidx < _NSTRIPE - 1))
        def _():
            h = _SW // 2
            pltpu.async_copy(
                tw_hbm.at[:, pl.ds(sidx * _SW, h)],
                bufs.at[b, :, pl.ds(0, h)], sems[b])
            pltpu.async_copy(
                tw_hbm.at[:, pl.ds(sidx * _SW + h, h)],
                bufs.at[b, :, pl.ds(h, h)], sems2[b])

        @pl.when((t < nstr_w) & (sidx == _NSTRIPE - 1))
        def _():
            pltpu.async_copy(
                tw_hbm.at[:, pl.ds(_LAST_W, _LAST_N)], tail_buf, sems[b])

    def wait(t, b):
        sidx = base + t

        @pl.when((t < nstr_w) & (sidx < _NSTRIPE - 1))
        def _():
            h = _SW // 2
            pltpu.make_async_copy(
                tw_hbm.at[:, pl.ds(sidx * _SW, h)],
                bufs.at[b, :, pl.ds(0, h)], sems[b]).wait()
            pltpu.make_async_copy(
                tw_hbm.at[:, pl.ds(sidx * _SW + h, h)],
                bufs.at[b, :, pl.ds(h, h)], sems2[b]).wait()

        @pl.when((t < nstr_w) & (sidx == _NSTRIPE - 1))
        def _():
            pltpu.make_async_copy(
                tw_hbm.at[:, pl.ds(_LAST_W, _LAST_N)], tail_buf,
                sems[b]).wait()

            def tcopy(r, z):
                for c in range(4):
                    bufs[b, r, pl.ds(c * 16, 16)] = tail_buf[r, pl.ds(c * 16, 16)]
                return z

            lax.fori_loop(0, 64, tcopy, 0)


    for b in range(_NBUF):
        fire(b, b)

    lo = base * _SW
    hi = (base + _SPW) * _SW

    for p in range(2):
        pltpu.async_copy(idx_hbm.at[pl.ds(p * _IW, _IW)], idxw.at[p],
                         isems[p])

    def prefilter_win(g, cnt):
        for p in range(2):
            win = g * 2 + p
            w0 = win * _IW
            pltpu.make_async_copy(idx_hbm.at[pl.ds(w0, _IW)], idxw.at[p],
                                  isems[p]).wait()

            def pf(c, cnt2):
                iv = idxw[p, pl.ds(c * 16, 16)]
                m = (iv >= lo) & (iv < hi)
                mc = plsc.all_reduce_population_count(m)[0]

                @pl.when(mc > 0)
                def _():
                    plsc.store_compressed(mi.at[pl.ds(cnt2, 16)], iv, mask=m)
                    plsc.store_compressed(mk.at[pl.ds(cnt2, 16)],
                                          w0 + c * 16 + lanes, mask=m)

                return cnt2 + mc

            cnt = lax.fori_loop(0, _IW // 16, pf, cnt)

            @pl.when(win + 2 < _B // _IW)
            def _():
                pltpu.async_copy(idx_hbm.at[pl.ds(w0 + 2 * _IW, _IW)],
                                 idxw.at[p], isems[p])

        return cnt

    cnt = lax.fori_loop(0, _B // _IW // 2, prefilter_win, 0)
    nchunk = (cnt + 15) // 16

    def process(t, b):
        @pl.when(t < nstr_w)
        def _():
            lo_s = (base + t) * _SW

            def extract_all():
                def extract(e, z):
                    eb = (e // 16) * 16
                    sel = lanes == (e - eb)
                    i_s = jnp.sum(jnp.where(sel, smi[pl.ds(eb, 16)], 0))
                    k_s = jnp.sum(jnp.where(sel, smk[pl.ds(eb, 16)], 0))
                    l_v = lax.rem(i_s, _SW) + jnp.zeros((16,), jnp.int32)
                    bj = cnts[0]
                    for c in range(4):
                        g = plsc.load_gather(bufs.at[b], [lanes + c * 16, l_v])
                        rows_v[bj, pl.ds(c * 16, 16)] = g
                    plsc.store_scatter(
                        klist, [bj + jnp.zeros((16,), jnp.int32)],
                        k_s + jnp.zeros((16,), jnp.int32), mask=lanes == 0)

                    @pl.when(bj == _BATCH - 1)
                    def _():
                        pltpu.async_copy(rows_v, outp_hbm.at[klist],
                                         sem_fl).wait()

                    cnts[0] = lax.rem(bj + 1, _BATCH)
                    return z

                lax.fori_loop(0, cnts[1], extract, 0)
                cnts[1] = 0

            cnts[1] = 0

            def scan(c, z):
                @pl.when(cnts[1] + 16 > _CAP)
                def _():
                    extract_all()

                iv = mi[pl.ds(c * 16, 16)]
                kv = mk[pl.ds(c * 16, 16)]
                m = (iv >= lo_s) & (iv < lo_s + _SW) & ((c * 16 + lanes) < cnt)
                mc = plsc.all_reduce_population_count(m)[0]
                sm = cnts[1]

                @pl.when(mc > 0)
                def _():
                    plsc.store_compressed(smi.at[pl.ds(sm, 16)], iv, mask=m)
                    plsc.store_compressed(smk.at[pl.ds(sm, 16)], kv, mask=m)

                cnts[1] = sm + mc
                return z

            lax.fori_loop(0, nchunk, scan, 0)
            extract_all()

    def outer(g, z):
        for b in range(_NBUF):
            t = g * _NBUF + b
            wait(t, b)
            process(t, b)
            fire(t + _NBUF, b)
        return z

    lax.fori_loop(0, _TRIPS, outer, 0)

    pltpu.async_copy(rows_v, outp_hbm.at[klist], sem_fl).wait()


def kernel(rel, rel_emb_weight):
    outp = _stream_gather(rel.astype(jnp.int32), rel_emb_weight.T)
    return outp[:_B, :_D]


# speedup vs baseline: 3.8720x; 1.0423x over previous
import functools

import jax
import jax.numpy as jnp
from jax import lax
from jax.experimental import pallas as pl
from jax.experimental.pallas import tpu as pltpu
from jax.experimental.pallas import tpu_sc as plsc

_B = 16384
_D = 64
_LANES = 1000000
_NC = 2
_NS = 16
_NW = _NC * _NS
_SW = 768
_NSTRIPE = (_LANES + _SW - 1) // _SW
_SPW = (_NSTRIPE + _NW - 1) // _NW
_NBUF = 2
_TRIPS = (_SPW + _NBUF - 1) // _NBUF
_LAST_W = (_NSTRIPE - 1) * _SW
_LAST_N = _LANES - _LAST_W
_BATCH = 32
_IW = 512
_CAP = 1024
_PK = 32768

_mesh = plsc.VectorSubcoreMesh(core_axis_name="c", subcore_axis_name="s")


@functools.partial(
    pl.kernel,
    mesh=_mesh,
    out_type=jax.ShapeDtypeStruct((_B + _BATCH, 128), jnp.float32),
    scratch_types=[
        pltpu.VMEM((2, _IW), jnp.int32),
        pltpu.VMEM((_B + 16,), jnp.int32),
        pltpu.VMEM((_CAP + 16,), jnp.int32),
        pltpu.VMEM((_NBUF, 64, _SW), jnp.float32),
        pltpu.VMEM((_BATCH, 128), jnp.float32),
        pltpu.VMEM((64, 64), jnp.float32),
        pltpu.VMEM((_BATCH,), jnp.int32),
        pltpu.SMEM((2,), jnp.int32),
        pltpu.SemaphoreType.DMA,
        pltpu.SemaphoreType.DMA,
        pltpu.SemaphoreType.DMA,
        pltpu.SemaphoreType.DMA,
        pltpu.SemaphoreType.DMA,
        pltpu.SemaphoreType.DMA,
        pltpu.SemaphoreType.DMA,
    ],
    compiler_params=pltpu.CompilerParams(needs_layout_passes=False),
)
def _stream_gather(idx_hbm, tw_hbm, outp_hbm, idxw, mp, sp, bufs, rows_v,
                   tail_buf, klist, cnts, sem_i0, sem_i1, sem_b0a, sem_b0b,
                   sem_b1a, sem_b1b, sem_fl):
    w = lax.axis_index("s") * _NC + lax.axis_index("c")
    base = w * _SPW
    nstr_w = jnp.minimum(base + _SPW, _NSTRIPE) - base
    sems = (sem_b0a, sem_b1a)
    sems2 = (sem_b0b, sem_b1b)
    isems = (sem_i0, sem_i1)
    lanes = lax.iota(jnp.int32, 16)

    for c in range(_BATCH // 16):
        klist[pl.ds(c * 16, 16)] = _B + c * 16 + lanes
    cnts[0] = 0

    lo = base * _SW
    hi = (base + _SPW) * _SW

    def fire(t, b):
        sidx = base + t

        @pl.when((t < nstr_w) & (sidx < _NSTRIPE - 1))
        def _():
            h = _SW // 2
            pltpu.async_copy(
                tw_hbm.at[:, pl.ds(sidx * _SW, h)],
                bufs.at[b, :, pl.ds(0, h)], sems[b])
            pltpu.async_copy(
                tw_hbm.at[:, pl.ds(sidx * _SW + h, h)],
                bufs.at[b, :, pl.ds(h, h)], sems2[b])

        @pl.when((t < nstr_w) & (sidx == _NSTRIPE - 1))
        def _():
            pltpu.async_copy(
                tw_hbm.at[:, pl.ds(_LAST_W, _LAST_N)], tail_buf, sems[b])

    def wait(t, b):
        sidx = base + t

        @pl.when((t < nstr_w) & (sidx < _NSTRIPE - 1))
        def _():
            h = _SW // 2
            pltpu.make_async_copy(
                tw_hbm.at[:, pl.ds(sidx * _SW, h)],
                bufs.at[b, :, pl.ds(0, h)], sems[b]).wait()
            pltpu.make_async_copy(
                tw_hbm.at[:, pl.ds(sidx * _SW + h, h)],
                bufs.at[b, :, pl.ds(h, h)], sems2[b]).wait()

        @pl.when((t < nstr_w) & (sidx == _NSTRIPE - 1))
        def _():
            pltpu.make_async_copy(
                tw_hbm.at[:, pl.ds(_LAST_W, _LAST_N)], tail_buf,
                sems[b]).wait()

            def tcopy(r, z):
                for c in range(4):
                    bufs[b, r, pl.ds(c * 16, 16)] = tail_buf[r, pl.ds(c * 16, 16)]
                return z

            lax.fori_loop(0, 64, tcopy, 0)

    for b in range(_NBUF):
        fire(b, b)

    for p in range(2):
        pltpu.async_copy(idx_hbm.at[pl.ds(p * _IW, _IW)], idxw.at[p],
                         isems[p])

    def prefilter_win(g, cnt):
        for p in range(2):
            win = g * 2 + p
            w0 = win * _IW
            pltpu.make_async_copy(idx_hbm.at[pl.ds(w0, _IW)], idxw.at[p],
                                  isems[p]).wait()

            def pf(c, cnt2):
                iv = idxw[p, pl.ds(c * 16, 16)]
                m = (iv >= lo) & (iv < hi)
                mc = plsc.all_reduce_population_count(m)[0]

                @pl.when(mc > 0)
                def _():
                    packed = (w0 + c * 16 + lanes) * _PK + (iv - lo)
                    plsc.store_compressed(mp.at[pl.ds(cnt2, 16)], packed,
                                          mask=m)

                return cnt2 + mc

            cnt = lax.fori_loop(0, _IW // 16, pf, cnt)

            @pl.when(win + 2 < _B // _IW)
            def _():
                pltpu.async_copy(idx_hbm.at[pl.ds(w0 + 2 * _IW, _IW)],
                                 idxw.at[p], isems[p])

        return cnt

    cnt = lax.fori_loop(0, _B // _IW // 2, prefilter_win, 0)
    nchunk = (cnt + 15) // 16

    def process(t, b):
        @pl.when(t < nstr_w)
        def _():
            rlo = t * _SW

            def extract_all():
                def extract(e, z):
                    eb = (e // 16) * 16
                    sel = lanes == (e - eb)
                    p_s = jnp.sum(jnp.where(sel, sp[pl.ds(eb, 16)], 0))
                    k_s = p_s // _PK
                    l_v = (p_s & (_PK - 1)) - rlo + jnp.zeros((16,), jnp.int32)
                    bj = cnts[0]
                    for c in range(4):
                        g = plsc.load_gather(bufs.at[b], [lanes + c * 16, l_v])
                        rows_v[bj, pl.ds(c * 16, 16)] = g
                    plsc.store_scatter(
                        klist, [bj + jnp.zeros((16,), jnp.int32)],
                        k_s + jnp.zeros((16,), jnp.int32), mask=lanes == 0)

                    @pl.when(bj == _BATCH - 1)
                    def _():
                        pltpu.async_copy(rows_v, outp_hbm.at[klist],
                                         sem_fl).wait()

                    cnts[0] = lax.rem(bj + 1, _BATCH)
                    return z

                lax.fori_loop(0, cnts[1], extract, 0)
                cnts[1] = 0

            cnts[1] = 0

            def scan(c, z):
                @pl.when(cnts[1] + 16 > _CAP)
                def _():
                    extract_all()

                mv = mp[pl.ds(c * 16, 16)]
                rel = mv & (_PK - 1)
                m = (rel >= rlo) & (rel < rlo + _SW) & ((c * 16 + lanes) < cnt)
                mc = plsc.all_reduce_population_count(m)[0]
                sm = cnts[1]

                @pl.when(mc > 0)
                def _():
                    plsc.store_compressed(sp.at[pl.ds(sm, 16)], mv, mask=m)

                cnts[1] = sm + mc
                return z

            lax.fori_loop(0, nchunk, scan, 0)
            extract_all()

    def outer(g, z):
        for b in range(_NBUF):
            t = g * _NBUF + b
            wait(t, b)
            process(t, b)
            fire(t + _NBUF, b)
        return z

    lax.fori_loop(0, _TRIPS, outer, 0)

    pltpu.async_copy(rows_v, outp_hbm.at[klist], sem_fl).wait()


def kernel(rel, rel_emb_weight):
    outp = _stream_gather(rel.astype(jnp.int32), rel_emb_weight.T)
    return outp[:_B, :_D]
